# baseline (speedup 1.0000x reference)
import functools

import jax
import jax.numpy as jnp
from jax import lax
from jax.experimental import pallas as pl
from jax.experimental.pallas import tpu as pltpu
from jax.experimental.pallas import tpu_sc as plsc

VOCAB_N = 100000
EMBED_D = 64
BATCH_B = 1024
CTX_W = 20

_NC = 2
_NS = 16
_NW = _NC * _NS
_ROWS_W = BATCH_B * CTX_W // _NW
_IDX_CH = _ROWS_W // 128
_B_W = BATCH_B // _NW


def _pool_body(xin_hbm, emb_hbm, out_hbm, idx_v, rows_v, acc_v, sem):
    wid = lax.axis_index("s") * _NC + lax.axis_index("c")
    for j in range(_IDX_CH):
        pltpu.sync_copy(
            xin_hbm.at[pl.ds(wid * _ROWS_W + j * 128, 128)], idx_v.at[j]
        )
    copies = [
        pltpu.async_copy(
            emb_hbm.at[idx_v.at[j]],
            rows_v.at[pl.ds(j * 128, 128)],
            sem,
        )
        for j in range(_IDX_CH)
    ]
    for c in copies:
        c.wait()

    def body(b, carry):
        r0 = b * CTX_W
        for k in range(EMBED_D // 16):
            acc = rows_v[r0, pl.ds(k * 16, 16)]
            for j in range(1, CTX_W):
                acc = acc + rows_v[r0 + j, pl.ds(k * 16, 16)]
            acc_v[b, pl.ds(k * 16, 16)] = acc
        return carry

    lax.fori_loop(0, _B_W, body, 0)
    pltpu.sync_copy(acc_v, out_hbm.at[pl.ds(wid * _B_W, _B_W)])


@functools.cache
def _build_pool():
    return pl.kernel(
        _pool_body,
        out_type=jax.ShapeDtypeStruct((BATCH_B, EMBED_D), jnp.float32),
        mesh=plsc.VectorSubcoreMesh(core_axis_name="c", subcore_axis_name="s"),
        scratch_types=[
            pltpu.VMEM((_IDX_CH, 128), jnp.int32),
            pltpu.VMEM((_ROWS_W, EMBED_D), jnp.float32),
            pltpu.VMEM((_B_W, EMBED_D), jnp.float32),
            pltpu.SemaphoreType.DMA,
        ],
        compiler_params=pltpu.CompilerParams(use_tc_tiling_on_sc=False),
    )


_N_BLK = 2048
_N_GRID = (VOCAB_N + _N_BLK - 1) // _N_BLK


def _mm_body(x_ref, w_ref, b_ref, o_ref):
    o_ref[...] = (
        lax.dot_general(
            x_ref[...].astype(jnp.bfloat16),
            w_ref[...].astype(jnp.bfloat16),
            (((1,), (1,)), ((), ())),
            preferred_element_type=jnp.float32,
        )
        + b_ref[...]
    )


def _matmul(x_sum, fc1_w, fc1_b2d):
    return pl.pallas_call(
        _mm_body,
        grid=(_N_GRID,),
        in_specs=[
            pl.BlockSpec((BATCH_B, EMBED_D), lambda i: (0, 0)),
            pl.BlockSpec((_N_BLK, EMBED_D), lambda i: (i, 0)),
            pl.BlockSpec((1, _N_BLK), lambda i: (0, i)),
        ],
        out_specs=pl.BlockSpec((BATCH_B, _N_BLK), lambda i: (0, i)),
        out_shape=jax.ShapeDtypeStruct((BATCH_B, VOCAB_N), jnp.float32),
        compiler_params=pltpu.CompilerParams(needs_layout_passes=True),
    )(x_sum, fc1_w, fc1_b2d)


def kernel(x_in, embedding, fc1_w, fc1_b):
    x_idx = x_in.astype(jnp.int32).reshape(BATCH_B * CTX_W)
    x_sum = _build_pool()(x_idx, embedding)
    return _matmul(x_sum, fc1_w, fc1_b.reshape(1, VOCAB_N))


# speedup vs baseline: 2.7435x; 2.7435x over previous
import functools

import jax
import jax.numpy as jnp
from jax import lax
from jax.experimental import pallas as pl
from jax.experimental.pallas import tpu as pltpu
from jax.experimental.pallas import tpu_sc as plsc

VOCAB_N = 100000
EMBED_D = 64
BATCH_B = 1024
CTX_W = 20

_NC = 2
_NS = 16
_NW = _NC * _NS
_ROWS_W = BATCH_B * CTX_W // _NW
_IDX_CH = _ROWS_W // 128
_B_W = BATCH_B // _NW


def _pool_body(xin_hbm, emb_hbm, out_hbm, idx_v, rows_v, acc_v, sem):
    wid = lax.axis_index("s") * _NC + lax.axis_index("c")
    for j in range(_IDX_CH):
        pltpu.sync_copy(
            xin_hbm.at[pl.ds(wid * _ROWS_W + j * 128, 128)], idx_v.at[j]
        )
    copies = [
        pltpu.async_copy(
            emb_hbm.at[idx_v.at[j]],
            rows_v.at[pl.ds(j * 128, 128)],
            sem,
        )
        for j in range(_IDX_CH)
    ]
    for c in copies:
        c.wait()

    def body(b, carry):
        r0 = b * CTX_W
        for k in range(EMBED_D // 16):
            acc = rows_v[r0, pl.ds(k * 16, 16)]
            for j in range(1, CTX_W):
                acc = acc + rows_v[r0 + j, pl.ds(k * 16, 16)]
            acc_v[b, pl.ds(k * 16, 16)] = acc
        return carry

    lax.fori_loop(0, _B_W, body, 0)
    pltpu.sync_copy(acc_v, out_hbm.at[pl.ds(wid * _B_W, _B_W)])


@functools.cache
def _build_pool():
    return pl.kernel(
        _pool_body,
        out_type=jax.ShapeDtypeStruct((BATCH_B, EMBED_D), jnp.float32),
        mesh=plsc.VectorSubcoreMesh(core_axis_name="c", subcore_axis_name="s"),
        scratch_types=[
            pltpu.VMEM((_IDX_CH, 128), jnp.int32),
            pltpu.VMEM((_ROWS_W, EMBED_D), jnp.float32),
            pltpu.VMEM((_B_W, EMBED_D), jnp.float32),
            pltpu.SemaphoreType.DMA,
        ],
        compiler_params=pltpu.CompilerParams(use_tc_tiling_on_sc=False),
    )


_N_BLK = 2048
_N_GRID = (VOCAB_N + _N_BLK - 1) // _N_BLK


def _mm_body(x_ref, wt_ref, b_ref, o_ref):
    yt = lax.dot_general(
        wt_ref[...].astype(jnp.bfloat16),
        x_ref[...].astype(jnp.bfloat16),
        (((0,), (1,)), ((), ())),
        preferred_element_type=jnp.float32,
    )
    o_ref[...] = yt + b_ref[...][:, None]


def _matmul_t(x_sum, fc1_wt, fc1_b):
    return pl.pallas_call(
        _mm_body,
        grid=(_N_GRID,),
        in_specs=[
            pl.BlockSpec((BATCH_B, EMBED_D), lambda i: (0, 0)),
            pl.BlockSpec((EMBED_D, _N_BLK), lambda i: (0, i)),
            pl.BlockSpec((_N_BLK,), lambda i: (i,)),
        ],
        out_specs=pl.BlockSpec((_N_BLK, BATCH_B), lambda i: (i, 0)),
        out_shape=jax.ShapeDtypeStruct((VOCAB_N, BATCH_B), jnp.float32),
    )(x_sum, fc1_wt, fc1_b)


def kernel(x_in, embedding, fc1_w, fc1_b):
    x_idx = x_in.astype(jnp.int32).reshape(BATCH_B * CTX_W)
    x_sum = _build_pool()(x_idx, embedding)
    return _matmul_t(x_sum, fc1_w.T, fc1_b).T
